# baseline (speedup 1.0000x reference)
import jax
import jax.numpy as jnp
from jax import lax
from jax.experimental import pallas as pl
from jax.experimental.pallas import tpu as pltpu
from jax.experimental.pallas import tpu_sc as plsc

N_H = 10
L = 16
NC, NS = 2, 16
NW = NC * NS
B, T = 4096, 256
BW = B // NW
GP = 32
NPASS = BW // GP

_TC = (0.9993386704758617, -0.3274132062807878, 0.1174902383200023,
       -0.03380254595095054, 0.00660837635036598, -0.0007449281113185158,
       3.58762642613808e-05)
_CLAMP = 2.25


def _tanh_poly(a, cs, clo, chi):
    a = jnp.minimum(jnp.maximum(a, clo), chi)
    c0, c1, c2, c3, c4, c5, c6 = cs
    u = a * a
    u2 = u * u
    u4 = u2 * u2
    p01 = c0 + c1 * u
    p23 = c2 + c3 * u
    p45 = c4 + c5 * u
    return a * (p01 + u2 * p23 + u4 * (p45 + u2 * c6))


def _tree_sum(prods):
    while len(prods) > 1:
        nxt = [prods[k] + prods[k + 1] for k in range(0, len(prods) - 1, 2)]
        if len(prods) % 2:
            nxt.append(prods[-1])
        prods = nxt
    return prods[0]


_GDN = lax.GatherDimensionNumbers(
    offset_dims=(), collapsed_slice_dims=(0,), start_index_map=(0,))
NWREG = N_H + 1


def _rnn_body(xT_hbm, wpack_hbm, out_hbm, x_v, out_v, wpack_v, h_v):
    wid = lax.axis_index("s") * NC + lax.axis_index("c")
    pltpu.sync_copy(wpack_hbm, wpack_v)
    pltpu.sync_copy(xT_hbm.at[:, pl.ds(wid * BW, BW)], x_v)

    iota = lax.iota(jnp.int32, L)
    row_even = iota * 2
    row_odd = iota * 2 + 1

    wregs = [wpack_v[r, :] for r in range(NWREG)]

    def wsplat(r, idx):
        w32 = lax.gather(wregs[r], idx, _GDN, (1,),
                         mode=lax.GatherScatterMode.PROMISE_IN_BOUNDS)
        return plsc.bitcast(w32, jnp.bfloat16)

    cs = tuple(jnp.full((2 * L,), c, jnp.bfloat16) for c in _TC)
    clo = jnp.full((2 * L,), -_CLAMP, jnp.bfloat16)
    chi = jnp.full((2 * L,), _CLAMP, jnp.bfloat16)

    def substep(h, xv, colbase):
        new_h = [None] * N_H
        for j in range(N_H):
            idx = jnp.full((L, 1), j, jnp.int32)
            prods = [xv * wsplat(N_H, idx)] + [h[i] * wsplat(i, idx)
                                               for i in range(N_H)]
            hv = _tanh_poly(_tree_sum(prods), cs, clo, chi)
            new_h[j] = hv
            ha, hb = plsc.unpack(hv, format=plsc.PackFormat.INTERLEAVED)
            col = jnp.broadcast_to(colbase + j, (L,))
            plsc.store_scatter(out_v, [row_even, col], ha)
            plsc.store_scatter(out_v, [row_odd, col], hb)
        return new_h

    def do_pass(p, carry):
        b0 = wid * BW + p * GP

        def step(t, h):
            return tuple(substep(list(h), x_v[t, pl.ds(p * GP, GP)],
                                 t * N_H))

        h0 = tuple(jnp.zeros((2 * L,), jnp.bfloat16) for _ in range(N_H))
        lax.fori_loop(0, T, step, h0, unroll=2)
        pltpu.sync_copy(out_v, out_hbm.at[pl.ds(b0, GP)])
        return carry

    lax.fori_loop(0, NPASS, do_pass, 0, unroll=False)


@jax.jit
def kernel(x, W_in, W_rec):
    xT = jnp.transpose(x).astype(jnp.bfloat16)
    w_mat = jnp.concatenate([W_rec, W_in[None, :]], axis=0)
    w_bf = jnp.pad(w_mat, ((0, 0), (0, L - N_H))).astype(jnp.bfloat16)
    w_u32 = lax.bitcast_convert_type(w_bf, jnp.uint16).astype(jnp.uint32)
    wpack = (w_u32 << 16) | w_u32

    run = pl.kernel(
        _rnn_body,
        out_type=jax.ShapeDtypeStruct((B, T * N_H), jnp.float32),
        mesh=plsc.VectorSubcoreMesh(core_axis_name="c", subcore_axis_name="s"),
        compiler_params=pltpu.CompilerParams(
            use_tc_tiling_on_sc=False, needs_layout_passes=False),
        scratch_types=[
            pltpu.VMEM((T, BW), jnp.bfloat16),
            pltpu.VMEM((GP, T * N_H), jnp.float32),
            pltpu.VMEM((NWREG, L), jnp.uint32),
            pltpu.VMEM((N_H, 2 * L), jnp.bfloat16),
        ],
    )
    return run(xT, wpack).reshape(B, T, N_H)


# speedup vs baseline: 1.0409x; 1.0409x over previous
import jax
import jax.numpy as jnp
from jax import lax
from jax.experimental import pallas as pl
from jax.experimental.pallas import tpu as pltpu
from jax.experimental.pallas import tpu_sc as plsc

N_H = 10
L = 16
NC, NS = 2, 16
NW = NC * NS
B, T = 4096, 256
BW = B // NW
GP = 2 * L
NG = BW // GP
CH = 32
NCH = T // CH
NWREG = N_H + 1

_TC = (0.9993386704758617, -0.3274132062807878, 0.1174902383200023,
       -0.03380254595095054, 0.00660837635036598, -0.0007449281113185158,
       3.58762642613808e-05)
_CLAMP = 2.25

_GDN = lax.GatherDimensionNumbers(
    offset_dims=(), collapsed_slice_dims=(0,), start_index_map=(0,))


def _tanh_poly(a, cs, clo, chi):
    a = jnp.minimum(jnp.maximum(a, clo), chi)
    c0, c1, c2, c3, c4, c5, c6 = cs
    u = a * a
    u2 = u * u
    u4 = u2 * u2
    p01 = c0 + c1 * u
    p23 = c2 + c3 * u
    p45 = c4 + c5 * u
    return a * (p01 + u2 * p23 + u4 * (p45 + u2 * c6))


def _tree_sum(prods):
    while len(prods) > 1:
        nxt = [prods[k] + prods[k + 1] for k in range(0, len(prods) - 1, 2)]
        if len(prods) % 2:
            nxt.append(prods[-1])
        prods = nxt
    return prods[0]


def _rnn_body(xT_hbm, wpack_hbm, out_hbm, x_v, out_v, wpack_v, ha_v, hb_v,
              sems):
    wid = lax.axis_index("s") * NC + lax.axis_index("c")
    pltpu.sync_copy(wpack_hbm, wpack_v)
    pltpu.sync_copy(xT_hbm.at[:, pl.ds(wid * BW, BW)], x_v)

    iota = lax.iota(jnp.int32, L)
    rows_e = [g * GP + iota * 2 for g in range(NG)]
    rows_o = [g * GP + iota * 2 + 1 for g in range(NG)]

    wregs = [wpack_v[r, :] for r in range(NWREG)]

    def wsplat(r, idx):
        w32 = lax.gather(wregs[r], idx, _GDN, (1,),
                         mode=lax.GatherScatterMode.PROMISE_IN_BOUNDS)
        return plsc.bitcast(w32, jnp.bfloat16)

    cs = tuple(jnp.full((GP,), c, jnp.bfloat16) for c in _TC)
    clo = jnp.full((GP,), -_CLAMP, jnp.bfloat16)
    chi = jnp.full((GP,), _CLAMP, jnp.bfloat16)

    zero = jnp.zeros((GP,), jnp.bfloat16)
    for k in range(NG * N_H):
        ha_v[k, :] = zero

    def hbm_slice(c):
        return out_hbm.at[pl.ds(wid * BW, BW), pl.ds(c * CH * N_H, CH * N_H)]

    def do_chunk(c, carry):
        @pl.when(c > 1)
        def _wait():
            pltpu.make_async_copy(out_v.at[c % 2], hbm_slice(c), sems.at[c % 2]
                                  ).wait()

        def substep(rbuf, wbuf, t, cb):
            xs = [x_v[t, pl.ds(g * GP, GP)] for g in range(NG)]
            ob = out_v.at[c % 2]
            for g0 in range(0, NG, 2):
                pair = (g0, g0 + 1)
                hvals = {(g, i): rbuf[g * N_H + i, :]
                         for g in pair for i in range(N_H)}
                for j in range(N_H):
                    idx = jnp.full((L, 1), j, jnp.int32)
                    ws = [wsplat(r, idx) for r in range(NWREG)]
                    col = jnp.broadcast_to(cb + j, (L,))
                    for g in pair:
                        prods = [xs[g] * ws[N_H]] + [
                            hvals[(g, i)] * ws[i] for i in range(N_H)]
                        hv = _tanh_poly(_tree_sum(prods), cs, clo, chi)
                        wbuf[g * N_H + j, :] = hv
                        ha, hb = plsc.unpack(
                            hv, format=plsc.PackFormat.INTERLEAVED)
                        plsc.store_scatter(ob, [rows_e[g], col], ha)
                        plsc.store_scatter(ob, [rows_o[g], col], hb)

        def step2(k, _):
            t0 = c * CH + 2 * k
            substep(ha_v, hb_v, t0, (2 * k) * N_H)
            substep(hb_v, ha_v, t0 + 1, (2 * k + 1) * N_H)
            return 0

        lax.fori_loop(0, CH // 2, step2, 0, unroll=False)
        pltpu.async_copy(out_v.at[c % 2], hbm_slice(c), sems.at[c % 2])
        return carry

    lax.fori_loop(0, NCH, do_chunk, 0, unroll=False)
    for c in (NCH - 2, NCH - 1):
        pltpu.make_async_copy(out_v.at[c % 2], hbm_slice(c), sems.at[c % 2]
                              ).wait()


@jax.jit
def kernel(x, W_in, W_rec):
    xT = jnp.transpose(x).astype(jnp.bfloat16)
    w_mat = jnp.concatenate([W_rec, W_in[None, :]], axis=0)
    w_bf = jnp.pad(w_mat, ((0, 0), (0, L - N_H))).astype(jnp.bfloat16)
    w_u32 = lax.bitcast_convert_type(w_bf, jnp.uint16).astype(jnp.uint32)
    wpack = (w_u32 << 16) | w_u32

    run = pl.kernel(
        _rnn_body,
        out_type=jax.ShapeDtypeStruct((B, T * N_H), jnp.float32),
        mesh=plsc.VectorSubcoreMesh(core_axis_name="c", subcore_axis_name="s"),
        compiler_params=pltpu.CompilerParams(
            use_tc_tiling_on_sc=False, needs_layout_passes=False),
        scratch_types=[
            pltpu.VMEM((T, BW), jnp.bfloat16),
            pltpu.VMEM((2, BW, CH * N_H), jnp.float32),
            pltpu.VMEM((NWREG, L), jnp.uint32),
            pltpu.VMEM((NG * N_H, GP), jnp.bfloat16),
            pltpu.VMEM((NG * N_H, GP), jnp.bfloat16),
            pltpu.SemaphoreType.DMA((2,)),
        ],
    )
    return run(xT, wpack).reshape(B, T, N_H)


# speedup vs baseline: 1.2910x; 1.2402x over previous
import jax
import jax.numpy as jnp
from jax import lax
from jax.experimental import pallas as pl
from jax.experimental.pallas import tpu as pltpu
from jax.experimental.pallas import tpu_sc as plsc

N_H = 10
L = 16
NC, NS = 2, 16
NW = NC * NS
B, T = 4096, 256
BW = B // NW
GP = 2 * L
NG = BW // GP
CH = 64
NCH = T // CH
NWREG = N_H + 1

_TC = (0.9995172394081107, -0.3284476622101936, 0.11875507424810447,
       -0.033856299279504266, 0.006042477113002057, -0.00047716912307449455)
_CLAMP = 1.8

_GDN = lax.GatherDimensionNumbers(
    offset_dims=(), collapsed_slice_dims=(0,), start_index_map=(0,))


def _tanh_poly(a, cs, clo, chi):
    a = jnp.minimum(jnp.maximum(a, clo), chi)
    c0, c1, c2, c3, c4, c5 = cs
    u = a * a
    u2 = u * u
    p01 = c0 + c1 * u
    p23 = c2 + c3 * u
    p45 = c4 + c5 * u
    return a * (p01 + u2 * p23 + (u2 * u2) * p45)


def _tree_sum(prods):
    while len(prods) > 1:
        nxt = [prods[k] + prods[k + 1] for k in range(0, len(prods) - 1, 2)]
        if len(prods) % 2:
            nxt.append(prods[-1])
        prods = nxt
    return prods[0]


def _rnn_body(xT_hbm, wpack_hbm, out_hbm, x_v, out_v, wpack_v, ha_v, hb_v,
              sems):
    wid = lax.axis_index("s") * NC + lax.axis_index("c")
    pltpu.sync_copy(wpack_hbm, wpack_v)
    pltpu.sync_copy(xT_hbm.at[:, pl.ds(wid * BW, BW)], x_v)

    iota = lax.iota(jnp.int32, L)
    rows_e = [g * GP + iota * 2 for g in range(2)]
    rows_o = [g * GP + iota * 2 + 1 for g in range(2)]

    wregs = [wpack_v[r, :] for r in range(NWREG)]

    def wsplat(r, idx):
        w32 = lax.gather(wregs[r], idx, _GDN, (1,),
                         mode=lax.GatherScatterMode.PROMISE_IN_BOUNDS)
        return plsc.bitcast(w32, jnp.bfloat16)

    cs = tuple(jnp.full((GP,), c, jnp.bfloat16) for c in _TC)
    clo = jnp.full((GP,), -_CLAMP, jnp.bfloat16)
    chi = jnp.full((GP,), _CLAMP, jnp.bfloat16)

    zero = jnp.zeros((GP,), jnp.bfloat16)
    for k in range(NG * N_H):
        ha_v[k, :] = zero

    def hbm_slice(c, p):
        return out_hbm.at[pl.ds(wid * BW + p * 2 * GP, 2 * GP),
                          pl.ds(c * CH * N_H, CH * N_H)]

    for p in range(2):
        pair = (2 * p, 2 * p + 1)

        def do_chunk(c, carry, p=p, pair=pair):
            @pl.when(c > 1)
            def _wait():
                pltpu.make_async_copy(out_v.at[c % 2], hbm_slice(c, p),
                                      sems.at[c % 2]).wait()

            def substep(rbuf, wbuf, t, cb):
                xs = {g: x_v[t, pl.ds(g * GP, GP)] for g in pair}
                ob = out_v.at[c % 2]
                hvals = {(g, i): rbuf[g * N_H + i, :]
                         for g in pair for i in range(N_H)}
                for j in range(N_H):
                    idx = jnp.full((L, 1), j, jnp.int32)
                    ws = [wsplat(r, idx) for r in range(NWREG)]
                    col = jnp.broadcast_to(cb + j, (L,))
                    for gl, g in enumerate(pair):
                        prods = [xs[g] * ws[N_H]] + [
                            hvals[(g, i)] * ws[i] for i in range(N_H)]
                        hv = _tanh_poly(_tree_sum(prods), cs, clo, chi)
                        wbuf[g * N_H + j, :] = hv
                        ha, hb = plsc.unpack(
                            hv, format=plsc.PackFormat.INTERLEAVED)
                        plsc.store_scatter(ob, [rows_e[gl], col], ha)
                        plsc.store_scatter(ob, [rows_o[gl], col], hb)

            def step2(k, _):
                t0 = c * CH + 2 * k
                substep(ha_v, hb_v, t0, (2 * k) * N_H)
                substep(hb_v, ha_v, t0 + 1, (2 * k + 1) * N_H)
                return 0

            lax.fori_loop(0, CH // 2, step2, 0, unroll=False)
            pltpu.async_copy(out_v.at[c % 2], hbm_slice(c, p),
                             sems.at[c % 2])
            return carry

        lax.fori_loop(0, NCH, do_chunk, 0, unroll=False)
        for c in (NCH - 2, NCH - 1):
            pltpu.make_async_copy(out_v.at[c % 2], hbm_slice(c, p),
                                  sems.at[c % 2]).wait()


@jax.jit
def kernel(x, W_in, W_rec):
    xT = jnp.transpose(x).astype(jnp.bfloat16)
    w_mat = jnp.concatenate([W_rec, W_in[None, :]], axis=0)
    w_bf = jnp.pad(w_mat, ((0, 0), (0, L - N_H))).astype(jnp.bfloat16)
    w_u32 = lax.bitcast_convert_type(w_bf, jnp.uint16).astype(jnp.uint32)
    wpack = (w_u32 << 16) | w_u32

    run = pl.kernel(
        _rnn_body,
        out_type=jax.ShapeDtypeStruct((B, T * N_H), jnp.float32),
        mesh=plsc.VectorSubcoreMesh(core_axis_name="c", subcore_axis_name="s"),
        compiler_params=pltpu.CompilerParams(
            use_tc_tiling_on_sc=False, needs_layout_passes=False),
        scratch_types=[
            pltpu.VMEM((T, BW), jnp.bfloat16),
            pltpu.VMEM((2, 2 * GP, CH * N_H), jnp.float32),
            pltpu.VMEM((NWREG, L), jnp.uint32),
            pltpu.VMEM((NG * N_H, GP), jnp.bfloat16),
            pltpu.VMEM((NG * N_H, GP), jnp.bfloat16),
            pltpu.SemaphoreType.DMA((2,)),
        ],
    )
    return run(xT, wpack).reshape(B, T, N_H)


# speedup vs baseline: 1.2936x; 1.0021x over previous
import jax
import jax.numpy as jnp
from jax import lax
from jax.experimental import pallas as pl
from jax.experimental.pallas import tpu as pltpu
from jax.experimental.pallas import tpu_sc as plsc

N_H = 10
L = 16
NC, NS = 2, 16
NW = NC * NS
B, T = 4096, 256
BW = B // NW
GP = 2 * L
NG = BW // GP
CH = 32
NCH = T // CH
NWREG = N_H + 1

_TC = (0.9995172394081107, -0.3284476622101936, 0.11875507424810447,
       -0.033856299279504266, 0.006042477113002057, -0.00047716912307449455)
_CLAMP = 1.8

_GDN = lax.GatherDimensionNumbers(
    offset_dims=(), collapsed_slice_dims=(0,), start_index_map=(0,))


def _tanh_poly(a, cs, clo, chi):
    a = jnp.minimum(jnp.maximum(a, clo), chi)
    c0, c1, c2, c3, c4, c5 = cs
    u = a * a
    u2 = u * u
    p01 = c0 + c1 * u
    p23 = c2 + c3 * u
    p45 = c4 + c5 * u
    return a * (p01 + u2 * p23 + (u2 * u2) * p45)


def _tree_sum(prods):
    while len(prods) > 1:
        nxt = [prods[k] + prods[k + 1] for k in range(0, len(prods) - 1, 2)]
        if len(prods) % 2:
            nxt.append(prods[-1])
        prods = nxt
    return prods[0]


def _rnn_body(xT_hbm, wpack_hbm, out_hbm, x_v, out_v, wpack_v, ha_v, hb_v,
              sems):
    wid = lax.axis_index("s") * NC + lax.axis_index("c")
    pltpu.sync_copy(wpack_hbm, wpack_v)
    pltpu.sync_copy(xT_hbm.at[:, pl.ds(wid * BW, BW)], x_v)

    iota = lax.iota(jnp.int32, L)
    rows_e = [g * GP + iota * 2 for g in range(2)]
    rows_o = [g * GP + iota * 2 + 1 for g in range(2)]

    wregs = [wpack_v[r, :] for r in range(NWREG)]

    def wsplat(r, idx):
        w32 = lax.gather(wregs[r], idx, _GDN, (1,),
                         mode=lax.GatherScatterMode.PROMISE_IN_BOUNDS)
        return plsc.bitcast(w32, jnp.bfloat16)

    cs = tuple(jnp.full((GP,), c, jnp.bfloat16) for c in _TC)
    clo = jnp.full((GP,), -_CLAMP, jnp.bfloat16)
    chi = jnp.full((GP,), _CLAMP, jnp.bfloat16)

    zero = jnp.zeros((GP,), jnp.bfloat16)
    for k in range(NG * N_H):
        ha_v[k, :] = zero

    def hbm_slice(c, p):
        return out_hbm.at[pl.ds(wid * BW + p * 2 * GP, 2 * GP),
                          pl.ds(c * CH * N_H, CH * N_H)]

    for p in range(2):
        pair = (2 * p, 2 * p + 1)

        def do_chunk(c, carry, p=p, pair=pair):
            @pl.when(c > 1)
            def _wait():
                pltpu.make_async_copy(out_v.at[c % 2], hbm_slice(c, p),
                                      sems.at[c % 2]).wait()

            def substep(rbuf, wbuf, t, cb):
                xs = {g: x_v[t, pl.ds(g * GP, GP)] for g in pair}
                ob = out_v.at[c % 2]
                hvals = {(g, i): rbuf[g * N_H + i, :]
                         for g in pair for i in range(N_H)}
                for j in range(N_H):
                    idx = jnp.full((L, 1), j, jnp.int32)
                    ws = [wsplat(r, idx) for r in range(NWREG)]
                    col = jnp.broadcast_to(cb + j, (L,))
                    for gl, g in enumerate(pair):
                        prods = [xs[g] * ws[N_H]] + [
                            hvals[(g, i)] * ws[i] for i in range(N_H)]
                        hv = _tanh_poly(_tree_sum(prods), cs, clo, chi)
                        wbuf[g * N_H + j, :] = hv
                        ha, hb = plsc.unpack(
                            hv, format=plsc.PackFormat.INTERLEAVED)
                        plsc.store_scatter(ob, [rows_e[gl], col], ha)
                        plsc.store_scatter(ob, [rows_o[gl], col], hb)

            def step2(k, _):
                t0 = c * CH + 2 * k
                substep(ha_v, hb_v, t0, (2 * k) * N_H)
                substep(hb_v, ha_v, t0 + 1, (2 * k + 1) * N_H)
                return 0

            lax.fori_loop(0, CH // 2, step2, 0, unroll=False)
            pltpu.async_copy(out_v.at[c % 2], hbm_slice(c, p),
                             sems.at[c % 2])
            return carry

        lax.fori_loop(0, NCH, do_chunk, 0, unroll=False)
        for c in (NCH - 2, NCH - 1):
            pltpu.make_async_copy(out_v.at[c % 2], hbm_slice(c, p),
                                  sems.at[c % 2]).wait()


@jax.jit
def kernel(x, W_in, W_rec):
    xT = jnp.transpose(x).astype(jnp.bfloat16)
    w_mat = jnp.concatenate([W_rec, W_in[None, :]], axis=0)
    w_bf = jnp.pad(w_mat, ((0, 0), (0, L - N_H))).astype(jnp.bfloat16)
    w_u32 = lax.bitcast_convert_type(w_bf, jnp.uint16).astype(jnp.uint32)
    wpack = (w_u32 << 16) | w_u32

    run = pl.kernel(
        _rnn_body,
        out_type=jax.ShapeDtypeStruct((B, T * N_H), jnp.float32),
        mesh=plsc.VectorSubcoreMesh(core_axis_name="c", subcore_axis_name="s"),
        compiler_params=pltpu.CompilerParams(
            use_tc_tiling_on_sc=False, needs_layout_passes=False),
        scratch_types=[
            pltpu.VMEM((T, BW), jnp.bfloat16),
            pltpu.VMEM((2, 2 * GP, CH * N_H), jnp.float32),
            pltpu.VMEM((NWREG, L), jnp.uint32),
            pltpu.VMEM((NG * N_H, GP), jnp.bfloat16),
            pltpu.VMEM((NG * N_H, GP), jnp.bfloat16),
            pltpu.SemaphoreType.DMA((2,)),
        ],
    )
    return run(xT, wpack).reshape(B, T, N_H)
